# Initial kernel scaffold; baseline (speedup 1.0000x reference)
import jax
import jax.numpy as jnp
from jax.experimental import pallas as pl


def kernel(h, p, coeffs, h_vals, p_vals):
    raise NotImplementedError("write your pallas kernel here")


# speedup vs baseline: 13.2398x; 13.2398x over previous
import functools

import jax
import jax.numpy as jnp
from jax import lax
from jax.experimental import pallas as pl
from jax.experimental.pallas import tpu as pltpu
from jax.experimental.pallas import tpu_sc as plsc

N_GRID = 511
NCELL = N_GRID * N_GRID
NPROP = 8
BATCH = 262144
NWORK = 32
QPW = BATCH // NWORK
CHUNK = 128
NCHUNK = QPW // CHUNK
NGRP = CHUNK // 16


def _prep_body(sc_ref, h_ref, p_ref, idx_ref, x_ref, y_ref):
    h = h_ref[...]
    p = p_ref[...]
    h_min = sc_ref[0]
    delta_h = sc_ref[1]
    logp_min = sc_ref[2]
    delta_logp = sc_ref[3]
    ii = (h - h_min) / delta_h
    jj = (jnp.log(p) - logp_min) / delta_logp
    i = jnp.clip(jnp.floor(ii).astype(jnp.int32), 0, N_GRID - 1)
    j = jnp.clip(jnp.floor(jj).astype(jnp.int32), 0, N_GRID - 1)
    idx_ref[...] = i * N_GRID + j
    x_ref[...] = ii - i.astype(jnp.float32)
    y_ref[...] = jj - j.astype(jnp.float32)


def _prep(scal, h2, p2):
    n = h2.shape[0]
    return pl.pallas_call(
        _prep_body,
        out_shape=(
            jax.ShapeDtypeStruct((n, 128), jnp.int32),
            jax.ShapeDtypeStruct((n, 128), jnp.float32),
            jax.ShapeDtypeStruct((n, 128), jnp.float32),
        ),
        in_specs=[
            pl.BlockSpec(memory_space=pltpu.SMEM),
            pl.BlockSpec((n, 128), lambda: (0, 0)),
            pl.BlockSpec((n, 128), lambda: (0, 0)),
        ],
        out_specs=(
            pl.BlockSpec((n, 128), lambda: (0, 0)),
            pl.BlockSpec((n, 128), lambda: (0, 0)),
            pl.BlockSpec((n, 128), lambda: (0, 0)),
        ),
    )(scal, h2, p2)


@functools.partial(
    pl.kernel,
    out_type=jax.ShapeDtypeStruct((NPROP, BATCH), jnp.float32),
    mesh=plsc.VectorSubcoreMesh(
        core_axis_name="c", subcore_axis_name="s", num_cores=2, num_subcores=16
    ),
    scratch_types=[
        pltpu.VMEM((CHUNK,), jnp.int32),
        pltpu.VMEM((CHUNK,), jnp.float32),
        pltpu.VMEM((CHUNK,), jnp.float32),
        pltpu.VMEM((CHUNK, 128), jnp.float32),
        pltpu.VMEM((NPROP * CHUNK,), jnp.float32),
        pltpu.SemaphoreType.DMA,
    ],
    compiler_params=pltpu.CompilerParams(
        needs_layout_passes=False, use_tc_tiling_on_sc=False
    ),
)
def _sc_main(tbl, idxq, xq, yq, out, idx_v, x_v, y_v, rows_v, o_v, sem):
    cid = lax.axis_index("c")
    sid = lax.axis_index("s")
    wid = sid * 2 + cid
    lane = lax.iota(jnp.int32, 16)

    def chunk_body(t, carry):
        base = wid * QPW + t * CHUNK
        pltpu.sync_copy(idxq.at[pl.ds(base, CHUNK)], idx_v)
        pltpu.sync_copy(xq.at[pl.ds(base, CHUNK)], x_v)
        pltpu.sync_copy(yq.at[pl.ds(base, CHUNK)], y_v)
        pltpu.async_copy(tbl.at[idx_v], rows_v, sem).wait()
        for g in range(NGRP):
            q0 = g * 16
            xv = x_v[pl.ds(q0, 16)]
            yv = y_v[pl.ds(q0, 16)]
            x2 = xv * xv
            x3 = x2 * xv
            y2 = yv * yv
            y3 = y2 * yv
            xs = (None, xv, x2, x3)
            ys = (None, yv, y2, y3)
            bas = []
            for ay in range(4):
                for ax in range(4):
                    if ay == 0:
                        bas.append(xs[ax])
                    elif ax == 0:
                        bas.append(ys[ay])
                    else:
                        bas.append(ys[ay] * xs[ax])
            qvec = lane + q0
            for prop in range(NPROP):
                acc = plsc.load_gather(
                    rows_v, [qvec, jnp.full((16,), prop * 16, jnp.int32)]
                )
                for k in range(1, 16):
                    kvec = jnp.full((16,), prop * 16 + k, jnp.int32)
                    gk = plsc.load_gather(rows_v, [qvec, kvec])
                    acc = acc + gk * bas[k]
                o_v[pl.ds(prop * CHUNK + q0, 16)] = acc
        for prop in range(NPROP):
            pltpu.sync_copy(
                o_v.at[pl.ds(prop * CHUNK, CHUNK)],
                out.at[prop, pl.ds(base, CHUNK)],
            )
        return carry

    lax.fori_loop(0, NCHUNK, chunk_body, 0)


def kernel(h, p, coeffs, h_vals, p_vals):
    h_min = h_vals[0]
    h_max = h_vals[-1]
    logp_min = jnp.log(p_vals[0])
    logp_max = jnp.log(p_vals[-1])
    delta_h = (h_max - h_min) / N_GRID
    delta_logp = (logp_max - logp_min) / N_GRID
    scal = jnp.stack([h_min, delta_h, logp_min, delta_logp])
    h2 = h.reshape(-1, 128)
    p2 = p.reshape(-1, 128)
    idx2, x2, y2 = _prep(scal, h2, p2)
    tbl = jnp.transpose(coeffs, (1, 2, 0, 3)).reshape(NCELL, NPROP * 16)
    out = _sc_main(tbl, idx2.reshape(-1), x2.reshape(-1), y2.reshape(-1))
    return out


# speedup vs baseline: 14.7806x; 1.1164x over previous
import functools

import jax
import jax.numpy as jnp
from jax import lax
from jax.experimental import pallas as pl
from jax.experimental.pallas import tpu as pltpu
from jax.experimental.pallas import tpu_sc as plsc

N_GRID = 511
NCELL = N_GRID * N_GRID
NPROP = 8
BATCH = 262144
NWORK = 32
QPW = BATCH // NWORK
CHUNK = 256
NCHUNK = QPW // CHUNK
NGRP = CHUNK // 16


def _prep_body(sc_ref, h_ref, p_ref, pk_ref):
    h = h_ref[...]
    p = p_ref[...]
    h_min = sc_ref[0]
    delta_h = sc_ref[1]
    logp_min = sc_ref[2]
    delta_logp = sc_ref[3]
    ii = (h - h_min) / delta_h
    jj = (jnp.log(p) - logp_min) / delta_logp
    i = jnp.clip(jnp.floor(ii).astype(jnp.int32), 0, N_GRID - 1)
    j = jnp.clip(jnp.floor(jj).astype(jnp.int32), 0, N_GRID - 1)
    idx = i * N_GRID + j
    x = ii - i.astype(jnp.float32)
    y = jj - j.astype(jnp.float32)
    pk_ref[...] = jnp.concatenate(
        [
            idx,
            jax.lax.bitcast_convert_type(x, jnp.int32),
            jax.lax.bitcast_convert_type(y, jnp.int32),
        ],
        axis=1,
    )


def _prep(scal, h2, p2):
    n = h2.shape[0]
    return pl.pallas_call(
        _prep_body,
        out_shape=jax.ShapeDtypeStruct((n, 384), jnp.int32),
        in_specs=[
            pl.BlockSpec(memory_space=pltpu.SMEM),
            pl.BlockSpec((n, 128), lambda: (0, 0)),
            pl.BlockSpec((n, 128), lambda: (0, 0)),
        ],
        out_specs=pl.BlockSpec((n, 384), lambda: (0, 0)),
    )(scal, h2, p2)


@functools.partial(
    pl.kernel,
    out_type=jax.ShapeDtypeStruct((NPROP, BATCH), jnp.float32),
    mesh=plsc.VectorSubcoreMesh(
        core_axis_name="c", subcore_axis_name="s", num_cores=2, num_subcores=16
    ),
    scratch_types=[
        pltpu.VMEM((2, 384), jnp.int32),
        pltpu.VMEM((CHUNK, 128), jnp.float32),
        pltpu.VMEM((NPROP, CHUNK), jnp.float32),
        pltpu.SemaphoreType.DMA,
    ],
    compiler_params=pltpu.CompilerParams(
        needs_layout_passes=False, use_tc_tiling_on_sc=False
    ),
)
def _sc_main(tbl, pk, out, in_v, rows_v, o_v, sem):
    cid = lax.axis_index("c")
    sid = lax.axis_index("s")
    wid = sid * 2 + cid
    lane = lax.iota(jnp.int32, 16)

    def chunk_body(t, carry):
        base = wid * QPW + t * CHUNK
        row0 = base // 128
        pltpu.sync_copy(pk.at[pl.ds(row0, 2)], in_v)
        d0 = pltpu.async_copy(
            tbl.at[in_v.at[0, pl.ds(0, 128)]], rows_v.at[pl.ds(0, 128)], sem
        )
        d1 = pltpu.async_copy(
            tbl.at[in_v.at[1, pl.ds(0, 128)]], rows_v.at[pl.ds(128, 128)], sem
        )
        d0.wait()
        d1.wait()
        for g in range(NGRP):
            sub = g // 8
            loc = (g % 8) * 16
            xv = plsc.bitcast(in_v[sub, pl.ds(128 + loc, 16)], jnp.float32)
            yv = plsc.bitcast(in_v[sub, pl.ds(256 + loc, 16)], jnp.float32)
            x2 = xv * xv
            x3 = x2 * xv
            y2 = yv * yv
            y3 = y2 * yv
            xs = (None, xv, x2, x3)
            ys = (None, yv, y2, y3)
            bas = []
            for ay in range(4):
                for ax in range(4):
                    if ay == 0:
                        bas.append(xs[ax])
                    elif ax == 0:
                        bas.append(ys[ay])
                    else:
                        bas.append(ys[ay] * xs[ax])
            qvec = lane + g * 16
            for prop in range(NPROP):
                acc = plsc.load_gather(
                    rows_v, [qvec, jnp.full((16,), prop * 16, jnp.int32)]
                )
                for k in range(1, 16):
                    kvec = jnp.full((16,), prop * 16 + k, jnp.int32)
                    gk = plsc.load_gather(rows_v, [qvec, kvec])
                    acc = acc + gk * bas[k]
                o_v[prop, pl.ds(g * 16, 16)] = acc
        pltpu.sync_copy(o_v, out.at[:, pl.ds(base, CHUNK)])
        return carry

    lax.fori_loop(0, NCHUNK, chunk_body, 0)


def kernel(h, p, coeffs, h_vals, p_vals):
    h_min = h_vals[0]
    h_max = h_vals[-1]
    logp_min = jnp.log(p_vals[0])
    logp_max = jnp.log(p_vals[-1])
    delta_h = (h_max - h_min) / N_GRID
    delta_logp = (logp_max - logp_min) / N_GRID
    scal = jnp.stack([h_min, delta_h, logp_min, delta_logp])
    h2 = h.reshape(-1, 128)
    p2 = p.reshape(-1, 128)
    pk = _prep(scal, h2, p2)
    tbl = jnp.transpose(coeffs, (1, 2, 0, 3)).reshape(NCELL, NPROP * 16)
    out = _sc_main(tbl, pk)
    return out
